# baseline (device time: 78160 ns/iter reference)
import jax
import jax.numpy as jnp
from jax import lax
from jax.experimental import pallas as pl
from jax.experimental.pallas import tpu as pltpu

N_DEV = 16
STEPS = (1, 2, 4, 8)
N_LAYERS = 3
TOTAL_STEPS = len(STEPS) * N_LAYERS


def kernel(x, Win0, Wout0, Win1, Wout1, Win2, Wout2):
    b, d = x.shape

    def body(x_ref, win0_ref, wout0_ref, win1_ref, wout1_ref,
             win2_ref, wout2_ref, out_ref, acc_ref, recv_ref,
             send_sems, recv_sems):
        my_i = lax.axis_index("i")

        def partner_of(dist):
            bit = (my_i // dist) % 2
            return my_i + dist - 2 * dist * bit

        barrier_sem = pltpu.get_barrier_semaphore()
        for dist in STEPS:
            pl.semaphore_signal(
                barrier_sem, inc=1,
                device_id=(partner_of(dist),),
                device_id_type=pl.DeviceIdType.MESH,
            )
        pl.semaphore_wait(barrier_sem, len(STEPS))

        step = 0
        x_cur = x_ref[...]
        for win_ref, wout_ref in ((win0_ref, wout0_ref),
                                  (win1_ref, wout1_ref),
                                  (win2_ref, wout2_ref)):
            h = jnp.maximum(
                jnp.dot(x_cur, win_ref[...], preferred_element_type=jnp.float32),
                0.0,
            )
            acc_ref[...] = jnp.dot(
                h, wout_ref[...], preferred_element_type=jnp.float32
            )
            for dist in STEPS:
                rdma = pltpu.make_async_remote_copy(
                    src_ref=acc_ref,
                    dst_ref=recv_ref.at[step],
                    send_sem=send_sems.at[step],
                    recv_sem=recv_sems.at[step],
                    device_id=(partner_of(dist),),
                    device_id_type=pl.DeviceIdType.MESH,
                )
                rdma.start()
                rdma.wait()
                acc_ref[...] = acc_ref[...] + recv_ref[step]
                step += 1
            x_cur = acc_ref[...]
        out_ref[...] = x_cur

    return pl.pallas_call(
        body,
        out_shape=jax.ShapeDtypeStruct((b, d), jnp.float32),
        in_specs=[pl.BlockSpec(memory_space=pltpu.VMEM)] * 7,
        out_specs=pl.BlockSpec(memory_space=pltpu.VMEM),
        scratch_shapes=[
            pltpu.VMEM((b, d), jnp.float32),
            pltpu.VMEM((TOTAL_STEPS, b, d), jnp.float32),
            pltpu.SemaphoreType.DMA((TOTAL_STEPS,)),
            pltpu.SemaphoreType.DMA((TOTAL_STEPS,)),
        ],
        compiler_params=pltpu.CompilerParams(collective_id=0),
    )(x, Win0, Wout0, Win1, Wout1, Win2, Wout2)


# device time: 62688 ns/iter; 1.2468x vs baseline; 1.2468x over previous
import jax
import jax.numpy as jnp
from jax import lax
from jax.experimental import pallas as pl
from jax.experimental.pallas import tpu as pltpu

N_DEV = 16
STEPS_A = (1, 2, 4, 8)
STEPS_B = (4, 8, 1, 2)
N_LAYERS = 3
TOTAL_STEPS = len(STEPS_A) * N_LAYERS


def kernel(x, Win0, Wout0, Win1, Wout1, Win2, Wout2):
    b, d = x.shape
    hb = b // 2

    def body(x_ref, win0_ref, wout0_ref, win1_ref, wout1_ref,
             win2_ref, wout2_ref, out_ref, acc_ref, recv_ref,
             send_sems, recv_sems):
        my_i = lax.axis_index("i")

        def partner_of(dist):
            bit = (my_i // dist) % 2
            return my_i + dist - 2 * dist * bit

        barrier_sem = pltpu.get_barrier_semaphore()
        for dist in STEPS_A:
            pl.semaphore_signal(
                barrier_sem, inc=1,
                device_id=(partner_of(dist),),
                device_id_type=pl.DeviceIdType.MESH,
            )
        pl.semaphore_wait(barrier_sem, len(STEPS_A))

        def exchange(step, half, dist):
            rows = pl.ds(half * hb, hb)
            rdma = pltpu.make_async_remote_copy(
                src_ref=acc_ref.at[rows, :],
                dst_ref=recv_ref.at[step, half],
                send_sem=send_sems.at[step, half],
                recv_sem=recv_sems.at[step, half],
                device_id=(partner_of(dist),),
                device_id_type=pl.DeviceIdType.MESH,
            )
            rdma.start()
            return rdma

        step = 0
        x_cur = x_ref[...]
        for win_ref, wout_ref in ((win0_ref, wout0_ref),
                                  (win1_ref, wout1_ref),
                                  (win2_ref, wout2_ref)):
            h = jnp.maximum(
                jnp.dot(x_cur, win_ref[...], preferred_element_type=jnp.float32),
                0.0,
            )
            acc_ref[...] = jnp.dot(
                h, wout_ref[...], preferred_element_type=jnp.float32
            )
            for da, db in zip(STEPS_A, STEPS_B):
                ra = exchange(step, 0, da)
                rb = exchange(step, 1, db)
                ra.wait()
                acc_ref[0:hb, :] = acc_ref[0:hb, :] + recv_ref[step, 0]
                rb.wait()
                acc_ref[hb:b, :] = acc_ref[hb:b, :] + recv_ref[step, 1]
                step += 1
            x_cur = acc_ref[...]
        out_ref[...] = x_cur

    return pl.pallas_call(
        body,
        out_shape=jax.ShapeDtypeStruct((b, d), jnp.float32),
        in_specs=[pl.BlockSpec(memory_space=pltpu.VMEM)] * 7,
        out_specs=pl.BlockSpec(memory_space=pltpu.VMEM),
        scratch_shapes=[
            pltpu.VMEM((b, d), jnp.float32),
            pltpu.VMEM((TOTAL_STEPS, 2, hb, d), jnp.float32),
            pltpu.SemaphoreType.DMA((TOTAL_STEPS, 2)),
            pltpu.SemaphoreType.DMA((TOTAL_STEPS, 2)),
        ],
        compiler_params=pltpu.CompilerParams(collective_id=0),
    )(x, Win0, Wout0, Win1, Wout1, Win2, Wout2)


# device time: 45734 ns/iter; 1.7090x vs baseline; 1.3707x over previous
import jax
import jax.numpy as jnp
from jax import lax
from jax.experimental import pallas as pl
from jax.experimental.pallas import tpu as pltpu

N_DEV = 16
N_LAYERS = 3


def kernel(x, Win0, Wout0, Win1, Wout1, Win2, Wout2):
    b, d = x.shape
    cb = b // N_DEV

    def body(x_ref, win0_ref, wout0_ref, win1_ref, wout1_ref,
             win2_ref, wout2_ref, out_ref, partial_ref, rs_ref,
             xnext_ref, send_sems, recv_sems):
        my_i = lax.axis_index("i")

        barrier_sem = pltpu.get_barrier_semaphore()
        for k in range(1, N_DEV):
            pl.semaphore_signal(
                barrier_sem, inc=1,
                device_id=((my_i + k) % N_DEV,),
                device_id_type=pl.DeviceIdType.MESH,
            )
        pl.semaphore_wait(barrier_sem, N_DEV - 1)

        def chunk_rows(i):
            return pl.ds(i * cb, cb)

        x_cur = x_ref[...]
        for win_ref, wout_ref in ((win0_ref, wout0_ref),
                                  (win1_ref, wout1_ref),
                                  (win2_ref, wout2_ref)):
            h = jnp.maximum(
                jnp.dot(x_cur, win_ref[...], preferred_element_type=jnp.float32),
                0.0,
            )
            partial_ref[...] = jnp.dot(
                h, wout_ref[...], preferred_element_type=jnp.float32
            )

            rs_sends = []
            for k in range(1, N_DEV):
                dest = (my_i + k) % N_DEV
                rdma = pltpu.make_async_remote_copy(
                    src_ref=partial_ref.at[chunk_rows(dest), :],
                    dst_ref=rs_ref.at[my_i],
                    send_sem=send_sems.at[0, k - 1],
                    recv_sem=recv_sems.at[0, k - 1],
                    device_id=(dest,),
                    device_id_type=pl.DeviceIdType.MESH,
                )
                rdma.start()
                rs_sends.append(rdma)

            for k in range(1, N_DEV):
                src = (my_i - k) % N_DEV
                pltpu.make_async_remote_copy(
                    src_ref=partial_ref.at[chunk_rows(my_i), :],
                    dst_ref=rs_ref.at[src],
                    send_sem=send_sems.at[0, k - 1],
                    recv_sem=recv_sems.at[0, k - 1],
                    device_id=(src,),
                    device_id_type=pl.DeviceIdType.MESH,
                ).wait_recv()

            reduced = partial_ref[chunk_rows(my_i), :]
            for k in range(1, N_DEV):
                reduced = reduced + rs_ref[(my_i + k) % N_DEV]
            xnext_ref[chunk_rows(my_i), :] = reduced

            ag_sends = []
            for k in range(1, N_DEV):
                dest = (my_i + k) % N_DEV
                rdma = pltpu.make_async_remote_copy(
                    src_ref=xnext_ref.at[chunk_rows(my_i), :],
                    dst_ref=xnext_ref.at[chunk_rows(my_i), :],
                    send_sem=send_sems.at[1, k - 1],
                    recv_sem=recv_sems.at[1, k - 1],
                    device_id=(dest,),
                    device_id_type=pl.DeviceIdType.MESH,
                )
                rdma.start()
                ag_sends.append(rdma)

            for k in range(1, N_DEV):
                src = (my_i - k) % N_DEV
                pltpu.make_async_remote_copy(
                    src_ref=xnext_ref.at[chunk_rows(src), :],
                    dst_ref=xnext_ref.at[chunk_rows(src), :],
                    send_sem=send_sems.at[1, k - 1],
                    recv_sem=recv_sems.at[1, k - 1],
                    device_id=(src,),
                    device_id_type=pl.DeviceIdType.MESH,
                ).wait_recv()

            for rdma in rs_sends:
                rdma.wait_send()
            for rdma in ag_sends:
                rdma.wait_send()

            x_cur = xnext_ref[...]
        out_ref[...] = x_cur

    return pl.pallas_call(
        body,
        out_shape=jax.ShapeDtypeStruct((b, d), jnp.float32),
        in_specs=[pl.BlockSpec(memory_space=pltpu.VMEM)] * 7,
        out_specs=pl.BlockSpec(memory_space=pltpu.VMEM),
        scratch_shapes=[
            pltpu.VMEM((b, d), jnp.float32),
            pltpu.VMEM((N_DEV, cb, d), jnp.float32),
            pltpu.VMEM((b, d), jnp.float32),
            pltpu.SemaphoreType.DMA((2, N_DEV - 1)),
            pltpu.SemaphoreType.DMA((2, N_DEV - 1)),
        ],
        compiler_params=pltpu.CompilerParams(collective_id=0),
    )(x, Win0, Wout0, Win1, Wout1, Win2, Wout2)


# device time: 45163 ns/iter; 1.7306x vs baseline; 1.0126x over previous
import jax
import jax.numpy as jnp
from jax import lax
from jax.experimental import pallas as pl
from jax.experimental.pallas import tpu as pltpu

N_DEV = 16
N_LAYERS = 3


def kernel(x, Win0, Wout0, Win1, Wout1, Win2, Wout2):
    b, d = x.shape
    cb = b // N_DEV

    def body(x_ref, win0_ref, wout0_ref, win1_ref, wout1_ref,
             win2_ref, wout2_ref, out_ref, partial_ref, rs_ref,
             xnext_ref, send_sems, recv_sems):
        my_i = lax.axis_index("i")

        barrier_sem = pltpu.get_barrier_semaphore()
        for k in range(1, N_DEV):
            pl.semaphore_signal(
                barrier_sem, inc=1,
                device_id=((my_i + k) % N_DEV,),
                device_id_type=pl.DeviceIdType.MESH,
            )

        def chunk_rows(i):
            return pl.ds(i * cb, cb)

        x_cur = x_ref[...]
        for layer, (win_ref, wout_ref) in enumerate(((win0_ref, wout0_ref),
                                                     (win1_ref, wout1_ref),
                                                     (win2_ref, wout2_ref))):
            h = jnp.maximum(
                jnp.dot(x_cur, win_ref[...], preferred_element_type=jnp.float32),
                0.0,
            )
            partial_ref[...] = jnp.dot(
                h, wout_ref[...], preferred_element_type=jnp.float32
            )
            if layer == 0:
                pl.semaphore_wait(barrier_sem, N_DEV - 1)

            rs_sends = []
            for k in range(1, N_DEV):
                dest = (my_i + k) % N_DEV
                rdma = pltpu.make_async_remote_copy(
                    src_ref=partial_ref.at[chunk_rows(dest), :],
                    dst_ref=rs_ref.at[my_i],
                    send_sem=send_sems.at[0, k - 1],
                    recv_sem=recv_sems.at[0, k - 1],
                    device_id=(dest,),
                    device_id_type=pl.DeviceIdType.MESH,
                )
                rdma.start()
                rs_sends.append(rdma)

            reduced = partial_ref[chunk_rows(my_i), :]
            for k in range(1, N_DEV):
                src = (my_i - k) % N_DEV
                pltpu.make_async_remote_copy(
                    src_ref=partial_ref.at[chunk_rows(my_i), :],
                    dst_ref=rs_ref.at[src],
                    send_sem=send_sems.at[0, k - 1],
                    recv_sem=recv_sems.at[0, k - 1],
                    device_id=(src,),
                    device_id_type=pl.DeviceIdType.MESH,
                ).wait_recv()
                reduced = reduced + rs_ref[(my_i - k) % N_DEV]
            xnext_ref[chunk_rows(my_i), :] = reduced

            ag_sends = []
            for k in range(1, N_DEV):
                dest = (my_i + k) % N_DEV
                rdma = pltpu.make_async_remote_copy(
                    src_ref=xnext_ref.at[chunk_rows(my_i), :],
                    dst_ref=xnext_ref.at[chunk_rows(my_i), :],
                    send_sem=send_sems.at[1, k - 1],
                    recv_sem=recv_sems.at[1, k - 1],
                    device_id=(dest,),
                    device_id_type=pl.DeviceIdType.MESH,
                )
                rdma.start()
                ag_sends.append(rdma)

            for k in range(1, N_DEV):
                src = (my_i - k) % N_DEV
                pltpu.make_async_remote_copy(
                    src_ref=xnext_ref.at[chunk_rows(src), :],
                    dst_ref=xnext_ref.at[chunk_rows(src), :],
                    send_sem=send_sems.at[1, k - 1],
                    recv_sem=recv_sems.at[1, k - 1],
                    device_id=(src,),
                    device_id_type=pl.DeviceIdType.MESH,
                ).wait_recv()

            for rdma in rs_sends:
                rdma.wait_send()
            for rdma in ag_sends:
                rdma.wait_send()

            x_cur = xnext_ref[...]
        out_ref[...] = x_cur

    return pl.pallas_call(
        body,
        out_shape=jax.ShapeDtypeStruct((b, d), jnp.float32),
        in_specs=[pl.BlockSpec(memory_space=pltpu.VMEM)] * 7,
        out_specs=pl.BlockSpec(memory_space=pltpu.VMEM),
        scratch_shapes=[
            pltpu.VMEM((b, d), jnp.float32),
            pltpu.VMEM((N_DEV, cb, d), jnp.float32),
            pltpu.VMEM((b, d), jnp.float32),
            pltpu.SemaphoreType.DMA((2, N_DEV - 1)),
            pltpu.SemaphoreType.DMA((2, N_DEV - 1)),
        ],
        compiler_params=pltpu.CompilerParams(collective_id=0),
    )(x, Win0, Wout0, Win1, Wout1, Win2, Wout2)
